# baseline (device time: 84271 ns/iter reference)
import jax
import jax.numpy as jnp
from jax import lax
from jax.experimental import pallas as pl
from jax.experimental.pallas import tpu as pltpu

N_DEV = 4
N_TOK = 1024
D_MODEL = 256
D_FF = 512
EXPERTS_PER_DEV = 4
CAPACITY = 51


def kernel(x, router_W, route_idx, expert_W):
    del router_W

    def body(x_ref, idx_ref, w_ref, out_ref, comm_ref, send_sems, recv_sems):
        my_pos = lax.axis_index("i")
        left = lax.rem(my_pos - 1 + N_DEV, N_DEV)
        right = lax.rem(my_pos + 1, N_DEV)

        barrier_sem = pltpu.get_barrier_semaphore()
        for nbr in [left, right]:
            pl.semaphore_signal(
                barrier_sem, inc=1,
                device_id=(nbr,), device_id_type=pl.DeviceIdType.MESH,
            )
        pl.semaphore_wait(barrier_sem, 2)

        r = idx_ref[:, :]
        local_e = my_pos * EXPERTS_PER_DEV + lax.broadcasted_iota(
            jnp.int32, (N_TOK, EXPERTS_PER_DEV), 1
        )
        match4 = (r == local_e).astype(jnp.float32)

        row = lax.broadcasted_iota(jnp.int32, (N_TOK, N_TOK), 0)
        col = lax.broadcasted_iota(jnp.int32, (N_TOK, N_TOK), 1)
        tri = (col <= row).astype(jnp.float32)
        cnt4 = jnp.dot(tri, match4, preferred_element_type=jnp.float32)
        keep4 = match4 * (cnt4 <= float(CAPACITY)).astype(jnp.float32)

        acc = jnp.zeros((N_TOK, D_FF), dtype=jnp.float32)
        for j in range(EXPERTS_PER_DEV):
            xm = x_ref[:, :] * keep4[:, j : j + 1]
            acc = acc + jnp.dot(
                xm, w_ref[j, :, :], preferred_element_type=jnp.float32
            )

        out_ref[:, :] = acc
        comm_ref[0, :, :] = acc

        for h in range(N_DEV - 1):
            send_slot = h % 2
            recv_slot = (h + 1) % 2
            rdma = pltpu.make_async_remote_copy(
                src_ref=comm_ref.at[send_slot],
                dst_ref=comm_ref.at[recv_slot],
                send_sem=send_sems.at[send_slot],
                recv_sem=recv_sems.at[recv_slot],
                device_id=(right,),
                device_id_type=pl.DeviceIdType.MESH,
            )
            rdma.start()
            rdma.wait()
            out_ref[:, :] += comm_ref[recv_slot, :, :]

    return pl.pallas_call(
        body,
        out_shape=jax.ShapeDtypeStruct((N_TOK, D_FF), jnp.float32),
        in_specs=[
            pl.BlockSpec(memory_space=pltpu.VMEM),
            pl.BlockSpec(memory_space=pltpu.VMEM),
            pl.BlockSpec(memory_space=pltpu.VMEM),
        ],
        out_specs=pl.BlockSpec(memory_space=pltpu.VMEM),
        scratch_shapes=[
            pltpu.VMEM((2, N_TOK, D_FF), jnp.float32),
            pltpu.SemaphoreType.DMA((2,)),
            pltpu.SemaphoreType.DMA((2,)),
        ],
        compiler_params=pltpu.CompilerParams(collective_id=0),
    )(x, route_idx, expert_W)


# device time: 37915 ns/iter; 2.2226x vs baseline; 2.2226x over previous
import jax
import jax.numpy as jnp
from jax import lax
from jax.experimental import pallas as pl
from jax.experimental.pallas import tpu as pltpu

N_DEV = 4
N_TOK = 1024
D_MODEL = 256
D_FF = 512
EXPERTS_PER_DEV = 4
CAPACITY = 51

CHUNK = N_TOK // N_DEV
HALF = D_FF // 2


def kernel(x, router_W, route_idx, expert_W):
    del router_W

    def body(x_ref, idx_ref, w_ref, out_ref,
             stage_a, stage_b, send_a, recv_a, send_b, recv_b):
        my_pos = lax.axis_index("i")
        left = lax.rem(my_pos + N_DEV - 1, N_DEV)
        right = lax.rem(my_pos + 1, N_DEV)

        barrier_sem = pltpu.get_barrier_semaphore()
        for nbr in [left, right]:
            pl.semaphore_signal(
                barrier_sem, inc=1,
                device_id=(nbr,), device_id_type=pl.DeviceIdType.MESH,
            )
        pl.semaphore_wait(barrier_sem, 2)

        r = idx_ref[:, :]
        local_e = my_pos * EXPERTS_PER_DEV + lax.broadcasted_iota(
            jnp.int32, (N_TOK, EXPERTS_PER_DEV), 1
        )
        match4 = (r == local_e).astype(jnp.float32)

        row = lax.broadcasted_iota(jnp.int32, (N_TOK, N_TOK), 0)
        col = lax.broadcasted_iota(jnp.int32, (N_TOK, N_TOK), 1)
        tri = (col <= row).astype(jnp.float32)
        cnt4 = jnp.dot(tri, match4, preferred_element_type=jnp.float32)
        keep4 = match4 * (cnt4 <= float(CAPACITY)).astype(jnp.float32)

        acc = jnp.zeros((N_TOK, D_FF), dtype=jnp.float32)
        for j in range(EXPERTS_PER_DEV):
            xm = x_ref[:, :] * keep4[:, j : j + 1]
            acc = acc + jnp.dot(
                xm, w_ref[j, :, :], preferred_element_type=jnp.float32
            )
        out_ref[:, :] = acc

        def rows(c):
            return pl.ds(c * CHUNK, CHUNK)

        def mod4(v):
            return lax.rem(v + 2 * N_DEV, N_DEV)

        cols_a = pl.ds(0, HALF)
        cols_b = pl.ds(HALF, HALF)

        for s in range(N_DEV - 1):
            slot = s % 2
            rdma_a = pltpu.make_async_remote_copy(
                src_ref=out_ref.at[rows(mod4(my_pos - s)), cols_a],
                dst_ref=stage_a.at[slot],
                send_sem=send_a.at[slot],
                recv_sem=recv_a.at[slot],
                device_id=(right,),
                device_id_type=pl.DeviceIdType.MESH,
            )
            rdma_b = pltpu.make_async_remote_copy(
                src_ref=out_ref.at[rows(mod4(my_pos + s)), cols_b],
                dst_ref=stage_b.at[slot],
                send_sem=send_b.at[slot],
                recv_sem=recv_b.at[slot],
                device_id=(left,),
                device_id_type=pl.DeviceIdType.MESH,
            )
            rdma_a.start()
            rdma_b.start()
            rdma_a.wait()
            rdma_b.wait()
            out_ref[rows(mod4(my_pos - s - 1)), cols_a] += stage_a[slot]
            out_ref[rows(mod4(my_pos + s + 1)), cols_b] += stage_b[slot]

        for s in range(N_DEV - 1):
            slot = (N_DEV - 1 + s) % 2
            ca = mod4(my_pos + 1 - s)
            cb = mod4(my_pos - 1 + s)
            rdma_a = pltpu.make_async_remote_copy(
                src_ref=out_ref.at[rows(ca), cols_a],
                dst_ref=out_ref.at[rows(ca), cols_a],
                send_sem=send_a.at[slot],
                recv_sem=recv_a.at[slot],
                device_id=(right,),
                device_id_type=pl.DeviceIdType.MESH,
            )
            rdma_b = pltpu.make_async_remote_copy(
                src_ref=out_ref.at[rows(cb), cols_b],
                dst_ref=out_ref.at[rows(cb), cols_b],
                send_sem=send_b.at[slot],
                recv_sem=recv_b.at[slot],
                device_id=(left,),
                device_id_type=pl.DeviceIdType.MESH,
            )
            rdma_a.start()
            rdma_b.start()
            rdma_a.wait()
            rdma_b.wait()

    return pl.pallas_call(
        body,
        out_shape=jax.ShapeDtypeStruct((N_TOK, D_FF), jnp.float32),
        in_specs=[
            pl.BlockSpec(memory_space=pltpu.VMEM),
            pl.BlockSpec(memory_space=pltpu.VMEM),
            pl.BlockSpec(memory_space=pltpu.VMEM),
        ],
        out_specs=pl.BlockSpec(memory_space=pltpu.VMEM),
        scratch_shapes=[
            pltpu.VMEM((2, CHUNK, HALF), jnp.float32),
            pltpu.VMEM((2, CHUNK, HALF), jnp.float32),
            pltpu.SemaphoreType.DMA((2,)),
            pltpu.SemaphoreType.DMA((2,)),
            pltpu.SemaphoreType.DMA((2,)),
            pltpu.SemaphoreType.DMA((2,)),
        ],
        compiler_params=pltpu.CompilerParams(collective_id=0),
    )(x, route_idx, expert_W)


# device time: 37910 ns/iter; 2.2229x vs baseline; 1.0001x over previous
import jax
import jax.numpy as jnp
from jax import lax
from jax.experimental import pallas as pl
from jax.experimental.pallas import tpu as pltpu

N_DEV = 4
N_TOK = 1024
D_MODEL = 256
D_FF = 512
EXPERTS_PER_DEV = 4
CAPACITY = 51

CHUNK = N_TOK // N_DEV
HALF = D_FF // 2


def kernel(x, router_W, route_idx, expert_W):
    del router_W

    def body(x_ref, idx_ref, w_ref, out_ref,
             stage_a, stage_b, send_a, recv_a, send_b, recv_b):
        my_pos = lax.axis_index("i")
        left = lax.rem(my_pos + N_DEV - 1, N_DEV)
        right = lax.rem(my_pos + 1, N_DEV)

        barrier_sem = pltpu.get_barrier_semaphore()
        for nbr in [left, right]:
            pl.semaphore_signal(
                barrier_sem, inc=1,
                device_id=(nbr,), device_id_type=pl.DeviceIdType.MESH,
            )
        pl.semaphore_wait(barrier_sem, 2)

        r = idx_ref[:, :]
        local_e = my_pos * EXPERTS_PER_DEV + lax.broadcasted_iota(
            jnp.int32, (N_TOK, EXPERTS_PER_DEV), 1
        )
        match4 = (r == local_e).astype(jnp.float32)

        row = lax.broadcasted_iota(jnp.int32, (N_TOK, N_TOK), 0)
        col = lax.broadcasted_iota(jnp.int32, (N_TOK, N_TOK), 1)
        tri = (col <= row).astype(jnp.float32)
        cnt4 = jnp.dot(tri, match4, preferred_element_type=jnp.float32)
        keep4 = match4 * (cnt4 <= float(CAPACITY)).astype(jnp.float32)

        acc = jnp.zeros((N_TOK, D_FF), dtype=jnp.float32)
        for j in range(EXPERTS_PER_DEV):
            xm = (x_ref[:, :] * keep4[:, j : j + 1]).astype(jnp.bfloat16)
            acc = acc + jnp.dot(
                xm,
                w_ref[j, :, :].astype(jnp.bfloat16),
                preferred_element_type=jnp.float32,
            )
        out_ref[:, :] = acc

        def rows(c):
            return pl.ds(c * CHUNK, CHUNK)

        def mod4(v):
            return lax.rem(v + 2 * N_DEV, N_DEV)

        cols_a = pl.ds(0, HALF)
        cols_b = pl.ds(HALF, HALF)

        for s in range(N_DEV - 1):
            slot = s % 2
            rdma_a = pltpu.make_async_remote_copy(
                src_ref=out_ref.at[rows(mod4(my_pos - s)), cols_a],
                dst_ref=stage_a.at[slot],
                send_sem=send_a.at[slot],
                recv_sem=recv_a.at[slot],
                device_id=(right,),
                device_id_type=pl.DeviceIdType.MESH,
            )
            rdma_b = pltpu.make_async_remote_copy(
                src_ref=out_ref.at[rows(mod4(my_pos + s)), cols_b],
                dst_ref=stage_b.at[slot],
                send_sem=send_b.at[slot],
                recv_sem=recv_b.at[slot],
                device_id=(left,),
                device_id_type=pl.DeviceIdType.MESH,
            )
            rdma_a.start()
            rdma_b.start()
            rdma_a.wait()
            rdma_b.wait()
            out_ref[rows(mod4(my_pos - s - 1)), cols_a] += stage_a[slot]
            out_ref[rows(mod4(my_pos + s + 1)), cols_b] += stage_b[slot]

        for s in range(N_DEV - 1):
            slot = (N_DEV - 1 + s) % 2
            ca = mod4(my_pos + 1 - s)
            cb = mod4(my_pos - 1 + s)
            rdma_a = pltpu.make_async_remote_copy(
                src_ref=out_ref.at[rows(ca), cols_a],
                dst_ref=out_ref.at[rows(ca), cols_a],
                send_sem=send_a.at[slot],
                recv_sem=recv_a.at[slot],
                device_id=(right,),
                device_id_type=pl.DeviceIdType.MESH,
            )
            rdma_b = pltpu.make_async_remote_copy(
                src_ref=out_ref.at[rows(cb), cols_b],
                dst_ref=out_ref.at[rows(cb), cols_b],
                send_sem=send_b.at[slot],
                recv_sem=recv_b.at[slot],
                device_id=(left,),
                device_id_type=pl.DeviceIdType.MESH,
            )
            rdma_a.start()
            rdma_b.start()
            rdma_a.wait()
            rdma_b.wait()

    return pl.pallas_call(
        body,
        out_shape=jax.ShapeDtypeStruct((N_TOK, D_FF), jnp.float32),
        in_specs=[
            pl.BlockSpec(memory_space=pltpu.VMEM),
            pl.BlockSpec(memory_space=pltpu.VMEM),
            pl.BlockSpec(memory_space=pltpu.VMEM),
        ],
        out_specs=pl.BlockSpec(memory_space=pltpu.VMEM),
        scratch_shapes=[
            pltpu.VMEM((2, CHUNK, HALF), jnp.float32),
            pltpu.VMEM((2, CHUNK, HALF), jnp.float32),
            pltpu.SemaphoreType.DMA((2,)),
            pltpu.SemaphoreType.DMA((2,)),
            pltpu.SemaphoreType.DMA((2,)),
            pltpu.SemaphoreType.DMA((2,)),
        ],
        compiler_params=pltpu.CompilerParams(collective_id=0),
    )(x, route_idx, expert_W)


# device time: 20620 ns/iter; 4.0869x vs baseline; 1.8385x over previous
import jax
import jax.numpy as jnp
from jax import lax
from jax.experimental import pallas as pl
from jax.experimental.pallas import tpu as pltpu

N_DEV = 4
N_TOK = 1024
D_MODEL = 256
D_FF = 512
EXPERTS_PER_DEV = 4
CAPACITY = 51
SLOT = 64
N_SLOTS = EXPERTS_PER_DEV * SLOT
HALF = D_FF // 2


def kernel(x, router_W, route_idx, expert_W):
    del router_W

    def body(x_ref, idx_ref, w_ref, out_ref,
             y_ref, buf_a, buf_b, send_a, recv_a, send_b, recv_b):
        my_pos = lax.axis_index("i")
        left = lax.rem(my_pos + N_DEV - 1, N_DEV)
        right = lax.rem(my_pos + 1, N_DEV)

        barrier_sem = pltpu.get_barrier_semaphore()
        for nbr in [left, right]:
            pl.semaphore_signal(
                barrier_sem, inc=1,
                device_id=(nbr,), device_id_type=pl.DeviceIdType.MESH,
            )
        pl.semaphore_wait(barrier_sem, 2)

        r = idx_ref[:, :]
        e16 = lax.broadcasted_iota(jnp.int32, (N_TOK, 16), 1)
        onehot = (r == e16).astype(jnp.bfloat16)
        row = lax.broadcasted_iota(jnp.int32, (N_TOK, N_TOK), 0)
        col = lax.broadcasted_iota(jnp.int32, (N_TOK, N_TOK), 1)
        tri = (col <= row).astype(jnp.bfloat16)
        cnt16 = jnp.dot(tri, onehot, preferred_element_type=jnp.float32)
        rk = jnp.sum(
            onehot.astype(jnp.float32) * cnt16, axis=1, keepdims=True
        )

        slot_j = lax.broadcasted_iota(jnp.int32, (N_TOK, N_SLOTS), 1) // SLOT
        slot_r = (
            lax.broadcasted_iota(jnp.int32, (N_TOK, N_SLOTS), 1) % SLOT + 1
        )
        slot_ok = (slot_r <= CAPACITY)

        def gather_t(d):
            slot_e = d * EXPERTS_PER_DEV + slot_j
            return (
                (r == slot_e) & (rk == slot_r.astype(jnp.float32)) & slot_ok
            ).astype(jnp.bfloat16)

        g_me = gather_t(my_pos)
        x_sel = lax.dot_general(
            g_me, x_ref[:, :].astype(jnp.bfloat16),
            dimension_numbers=(((0,), (0,)), ((), ())),
            preferred_element_type=jnp.float32,
        )
        for j in range(EXPERTS_PER_DEV):
            y_ref[j * SLOT : (j + 1) * SLOT, :] = jnp.dot(
                x_sel[j * SLOT : (j + 1) * SLOT, :].astype(jnp.bfloat16),
                w_ref[j, :, :].astype(jnp.bfloat16),
                preferred_element_type=jnp.float32,
            ).astype(jnp.bfloat16)

        cols_a = pl.ds(0, HALF)
        cols_b = pl.ds(HALF, HALF)

        def hop(h):
            return (
                pltpu.make_async_remote_copy(
                    src_ref=(y_ref.at[:, cols_a] if h == 0
                             else buf_a.at[h - 1]),
                    dst_ref=buf_a.at[h],
                    send_sem=send_a.at[h],
                    recv_sem=recv_a.at[h],
                    device_id=(right,),
                    device_id_type=pl.DeviceIdType.MESH,
                ),
                pltpu.make_async_remote_copy(
                    src_ref=(y_ref.at[:, cols_b] if h == 0
                             else buf_b.at[h - 1]),
                    dst_ref=buf_b.at[h],
                    send_sem=send_b.at[h],
                    recv_sem=recv_b.at[h],
                    device_id=(left,),
                    device_id_type=pl.DeviceIdType.MESH,
                ),
            )

        rdma_a, rdma_b = hop(0)
        rdma_a.start()
        rdma_b.start()

        out_ref[:, :] = jnp.dot(
            g_me, y_ref[:, :], preferred_element_type=jnp.float32
        )

        def mod4(v):
            return lax.rem(v + 2 * N_DEV, N_DEV)

        for h in range(N_DEV - 1):
            rdma_a.wait()
            rdma_b.wait()
            if h < N_DEV - 2:
                rdma_a, rdma_b = hop(h + 1)
                rdma_a.start()
                rdma_b.start()
            g_a = gather_t(mod4(my_pos - 1 - h))
            out_ref[:, cols_a] += jnp.dot(
                g_a, buf_a[h, :, :], preferred_element_type=jnp.float32
            )
            g_b = gather_t(mod4(my_pos + 1 + h))
            out_ref[:, cols_b] += jnp.dot(
                g_b, buf_b[h, :, :], preferred_element_type=jnp.float32
            )

    return pl.pallas_call(
        body,
        out_shape=jax.ShapeDtypeStruct((N_TOK, D_FF), jnp.float32),
        in_specs=[
            pl.BlockSpec(memory_space=pltpu.VMEM),
            pl.BlockSpec(memory_space=pltpu.VMEM),
            pl.BlockSpec(memory_space=pltpu.VMEM),
        ],
        out_specs=pl.BlockSpec(memory_space=pltpu.VMEM),
        scratch_shapes=[
            pltpu.VMEM((N_SLOTS, D_FF), jnp.bfloat16),
            pltpu.VMEM((N_DEV - 1, N_SLOTS, HALF), jnp.bfloat16),
            pltpu.VMEM((N_DEV - 1, N_SLOTS, HALF), jnp.bfloat16),
            pltpu.SemaphoreType.DMA((N_DEV - 1,)),
            pltpu.SemaphoreType.DMA((N_DEV - 1,)),
            pltpu.SemaphoreType.DMA((N_DEV - 1,)),
            pltpu.SemaphoreType.DMA((N_DEV - 1,)),
        ],
        compiler_params=pltpu.CompilerParams(collective_id=0),
    )(x, route_idx, expert_W)


# device time: 18701 ns/iter; 4.5062x vs baseline; 1.1026x over previous
import jax
import jax.numpy as jnp
from jax import lax
from jax.experimental import pallas as pl
from jax.experimental.pallas import tpu as pltpu

N_DEV = 4
N_TOK = 1024
D_MODEL = 256
D_FF = 512
EXPERTS_PER_DEV = 4
CAPACITY = 51
SLOT = 64
N_SLOTS = EXPERTS_PER_DEV * SLOT
HALF = D_FF // 2


def kernel(x, router_W, route_idx, expert_W):
    del router_W

    def body(x_ref, idx_ref, w_ref, out_ref,
             y_ref, buf_in, send_sems, recv_sems):
        my_pos = lax.axis_index("i")
        left = lax.rem(my_pos + N_DEV - 1, N_DEV)
        right = lax.rem(my_pos + 1, N_DEV)
        diag = lax.rem(my_pos + 2, N_DEV)

        barrier_sem = pltpu.get_barrier_semaphore()
        for nbr in [left, right, diag]:
            pl.semaphore_signal(
                barrier_sem, inc=1,
                device_id=(nbr,), device_id_type=pl.DeviceIdType.MESH,
            )
        pl.semaphore_wait(barrier_sem, 3)

        r = idx_ref[:, :]
        e16 = lax.broadcasted_iota(jnp.int32, (N_TOK, 16), 1)
        onehot = (r == e16).astype(jnp.bfloat16)
        row = lax.broadcasted_iota(jnp.int32, (N_TOK, N_TOK), 0)
        col = lax.broadcasted_iota(jnp.int32, (N_TOK, N_TOK), 1)
        tri = (col <= row).astype(jnp.bfloat16)
        cnt16 = jnp.dot(tri, onehot, preferred_element_type=jnp.float32)
        rk = jnp.sum(
            onehot.astype(jnp.float32) * cnt16, axis=1, keepdims=True
        )

        slot_j = lax.broadcasted_iota(jnp.int32, (N_TOK, N_SLOTS), 1) // SLOT
        slot_r = (
            lax.broadcasted_iota(jnp.int32, (N_TOK, N_SLOTS), 1) % SLOT + 1
        )
        slot_ok = (slot_r <= CAPACITY)

        def gather_t(d):
            slot_e = d * EXPERTS_PER_DEV + slot_j
            return (
                (r == slot_e) & (rk == slot_r.astype(jnp.float32)) & slot_ok
            ).astype(jnp.bfloat16)

        g_me = gather_t(my_pos)
        x_sel = lax.dot_general(
            g_me, x_ref[:, :].astype(jnp.bfloat16),
            dimension_numbers=(((0,), (0,)), ((), ())),
            preferred_element_type=jnp.float32,
        )
        for j in range(EXPERTS_PER_DEV):
            y_ref[j * SLOT : (j + 1) * SLOT, :] = jnp.dot(
                x_sel[j * SLOT : (j + 1) * SLOT, :].astype(jnp.bfloat16),
                w_ref[j, :, :].astype(jnp.bfloat16),
                preferred_element_type=jnp.float32,
            ).astype(jnp.bfloat16)

        def push(target, slot):
            rdma = pltpu.make_async_remote_copy(
                src_ref=y_ref,
                dst_ref=buf_in.at[slot],
                send_sem=send_sems.at[slot],
                recv_sem=recv_sems.at[slot],
                device_id=(target,),
                device_id_type=pl.DeviceIdType.MESH,
            )
            rdma.start()
            return rdma

        sends = [push(right, 0), push(left, 1), push(diag, 2)]

        out_ref[:, :] = jnp.dot(
            g_me, y_ref[:, :], preferred_element_type=jnp.float32
        )

        for slot, src in [(0, left), (1, right), (2, diag)]:
            sends[slot].wait_recv()
            g_src = gather_t(src)
            out_ref[:, :] += jnp.dot(
                g_src, buf_in[slot, :, :], preferred_element_type=jnp.float32
            )
        for s in sends:
            s.wait_send()

    return pl.pallas_call(
        body,
        out_shape=jax.ShapeDtypeStruct((N_TOK, D_FF), jnp.float32),
        in_specs=[
            pl.BlockSpec(memory_space=pltpu.VMEM),
            pl.BlockSpec(memory_space=pltpu.VMEM),
            pl.BlockSpec(memory_space=pltpu.VMEM),
        ],
        out_specs=pl.BlockSpec(memory_space=pltpu.VMEM),
        scratch_shapes=[
            pltpu.VMEM((N_SLOTS, D_FF), jnp.bfloat16),
            pltpu.VMEM((N_DEV - 1, N_SLOTS, D_FF), jnp.bfloat16),
            pltpu.SemaphoreType.DMA((N_DEV - 1,)),
            pltpu.SemaphoreType.DMA((N_DEV - 1,)),
        ],
        compiler_params=pltpu.CompilerParams(collective_id=0),
    )(x, route_idx, expert_W)


# device time: 17539 ns/iter; 4.8048x vs baseline; 1.0663x over previous
import jax
import jax.numpy as jnp
from jax import lax
from jax.experimental import pallas as pl
from jax.experimental.pallas import tpu as pltpu

N_DEV = 4
N_TOK = 1024
D_MODEL = 256
D_FF = 512
EXPERTS_PER_DEV = 4
CAPACITY = 51
SLOT = 52
N_SLOTS = EXPERTS_PER_DEV * SLOT
HALF = D_FF // 2


def kernel(x, router_W, route_idx, expert_W):
    del router_W

    def body(x_ref, idx_ref, w_ref, out_ref,
             y_ref, buf_in, send_sems, recv_sems):
        my_pos = lax.axis_index("i")
        left = lax.rem(my_pos + N_DEV - 1, N_DEV)
        right = lax.rem(my_pos + 1, N_DEV)
        diag = lax.rem(my_pos + 2, N_DEV)

        barrier_sem = pltpu.get_barrier_semaphore()
        for nbr in [left, right, diag]:
            pl.semaphore_signal(
                barrier_sem, inc=1,
                device_id=(nbr,), device_id_type=pl.DeviceIdType.MESH,
            )
        pl.semaphore_wait(barrier_sem, 3)

        r = idx_ref[:, :]
        e16 = lax.broadcasted_iota(jnp.int32, (N_TOK, 16), 1)
        onehot = (r == e16).astype(jnp.bfloat16)
        row = lax.broadcasted_iota(jnp.int32, (N_TOK, N_TOK), 0)
        col = lax.broadcasted_iota(jnp.int32, (N_TOK, N_TOK), 1)
        tri = (col <= row).astype(jnp.bfloat16)
        cnt16 = jnp.dot(tri, onehot, preferred_element_type=jnp.float32)
        rk = jnp.sum(
            onehot.astype(jnp.float32) * cnt16, axis=1, keepdims=True
        )

        slot_j = lax.broadcasted_iota(jnp.int32, (N_TOK, N_SLOTS), 1) // SLOT
        slot_r = (
            lax.broadcasted_iota(jnp.int32, (N_TOK, N_SLOTS), 1) % SLOT + 1
        )
        slot_ok = (slot_r <= CAPACITY)

        def gather_t(d):
            slot_e = d * EXPERTS_PER_DEV + slot_j
            return (
                (r == slot_e) & (rk == slot_r.astype(jnp.float32)) & slot_ok
            ).astype(jnp.bfloat16)

        g_me = gather_t(my_pos)
        x_sel = lax.dot_general(
            g_me, x_ref[:, :].astype(jnp.bfloat16),
            dimension_numbers=(((0,), (0,)), ((), ())),
            preferred_element_type=jnp.float32,
        )
        y_ref[:, :] = jnp.concatenate(
            [
                jnp.dot(
                    x_sel[j * SLOT : (j + 1) * SLOT, :].astype(jnp.bfloat16),
                    w_ref[j, :, :].astype(jnp.bfloat16),
                    preferred_element_type=jnp.float32,
                ).astype(jnp.bfloat16)
                for j in range(EXPERTS_PER_DEV)
            ],
            axis=0,
        )

        def push(target, slot):
            rdma = pltpu.make_async_remote_copy(
                src_ref=y_ref,
                dst_ref=buf_in.at[slot],
                send_sem=send_sems.at[slot],
                recv_sem=recv_sems.at[slot],
                device_id=(target,),
                device_id_type=pl.DeviceIdType.MESH,
            )
            rdma.start()
            return rdma

        sends = [push(right, 0), push(left, 1), push(diag, 2)]

        out_ref[:, :] = jnp.dot(
            g_me, y_ref[:, :], preferred_element_type=jnp.float32
        )

        for slot, src in [(0, left), (1, right), (2, diag)]:
            sends[slot].wait_recv()
            g_src = gather_t(src)
            out_ref[:, :] += jnp.dot(
                g_src, buf_in[slot, :, :], preferred_element_type=jnp.float32
            )
        for s in sends:
            s.wait_send()

    return pl.pallas_call(
        body,
        out_shape=jax.ShapeDtypeStruct((N_TOK, D_FF), jnp.float32),
        in_specs=[
            pl.BlockSpec(memory_space=pltpu.VMEM),
            pl.BlockSpec(memory_space=pltpu.VMEM),
            pl.BlockSpec(memory_space=pltpu.VMEM),
        ],
        out_specs=pl.BlockSpec(memory_space=pltpu.VMEM),
        scratch_shapes=[
            pltpu.VMEM((N_SLOTS, D_FF), jnp.bfloat16),
            pltpu.VMEM((N_DEV - 1, N_SLOTS, D_FF), jnp.bfloat16),
            pltpu.SemaphoreType.DMA((N_DEV - 1,)),
            pltpu.SemaphoreType.DMA((N_DEV - 1,)),
        ],
        compiler_params=pltpu.CompilerParams(collective_id=0),
    )(x, route_idx, expert_W)


# device time: 17097 ns/iter; 4.9290x vs baseline; 1.0259x over previous
import jax
import jax.numpy as jnp
from jax import lax
from jax.experimental import pallas as pl
from jax.experimental.pallas import tpu as pltpu

N_DEV = 4
N_TOK = 1024
D_MODEL = 256
D_FF = 512
EXPERTS_PER_DEV = 4
CAPACITY = 51
SLOT = 52
N_SLOTS = EXPERTS_PER_DEV * SLOT
HALF = D_FF // 2


def kernel(x, router_W, route_idx, expert_W):
    del router_W

    def body(x_ref, idx_ref, w_ref, out_ref,
             y_ref, buf_in, send_sems, recv_sems):
        my_pos = lax.axis_index("i")
        left = lax.rem(my_pos + N_DEV - 1, N_DEV)
        right = lax.rem(my_pos + 1, N_DEV)
        diag = lax.rem(my_pos + 2, N_DEV)

        barrier_sem = pltpu.get_barrier_semaphore()
        for nbr in [left, right, diag]:
            pl.semaphore_signal(
                barrier_sem, inc=1,
                device_id=(nbr,), device_id_type=pl.DeviceIdType.MESH,
            )

        r = idx_ref[:, :]
        e16 = lax.broadcasted_iota(jnp.int32, (N_TOK, 16), 1)
        onehot = (r == e16).astype(jnp.bfloat16)
        row = lax.broadcasted_iota(jnp.int32, (N_TOK, N_TOK), 0)
        col = lax.broadcasted_iota(jnp.int32, (N_TOK, N_TOK), 1)
        tri = (col <= row).astype(jnp.bfloat16)
        cnt16 = jnp.dot(tri, onehot, preferred_element_type=jnp.float32)
        rk = jnp.sum(
            onehot.astype(jnp.float32) * cnt16, axis=1, keepdims=True
        )

        slot_j = lax.broadcasted_iota(jnp.int32, (N_TOK, N_SLOTS), 1) // SLOT
        slot_r = (
            lax.broadcasted_iota(jnp.int32, (N_TOK, N_SLOTS), 1) % SLOT + 1
        )
        slot_ok = (slot_r <= CAPACITY)

        def gather_t(d):
            slot_e = d * EXPERTS_PER_DEV + slot_j
            return (
                (r == slot_e) & (rk == slot_r.astype(jnp.float32)) & slot_ok
            ).astype(jnp.bfloat16)

        g_me = gather_t(my_pos)
        x_sel = lax.dot_general(
            g_me, x_ref[:, :].astype(jnp.bfloat16),
            dimension_numbers=(((0,), (0,)), ((), ())),
            preferred_element_type=jnp.float32,
        )
        y_ref[:, :] = jnp.concatenate(
            [
                jnp.dot(
                    x_sel[j * SLOT : (j + 1) * SLOT, :].astype(jnp.bfloat16),
                    w_ref[j, :, :].astype(jnp.bfloat16),
                    preferred_element_type=jnp.float32,
                ).astype(jnp.bfloat16)
                for j in range(EXPERTS_PER_DEV)
            ],
            axis=0,
        )

        def push(target, slot):
            rdma = pltpu.make_async_remote_copy(
                src_ref=y_ref,
                dst_ref=buf_in.at[slot],
                send_sem=send_sems.at[slot],
                recv_sem=recv_sems.at[slot],
                device_id=(target,),
                device_id_type=pl.DeviceIdType.MESH,
            )
            rdma.start()
            return rdma

        pl.semaphore_wait(barrier_sem, 3)
        sends = [push(right, 0), push(left, 1), push(diag, 2)]

        out_ref[:, :] = jnp.dot(
            g_me, y_ref[:, :], preferred_element_type=jnp.float32
        )
        g_peers = [gather_t(left), gather_t(right), gather_t(diag)]

        for slot in range(3):
            sends[slot].wait_recv()
            out_ref[:, :] += jnp.dot(
                g_peers[slot], buf_in[slot, :, :],
                preferred_element_type=jnp.float32,
            )
        for s in sends:
            s.wait_send()

    return pl.pallas_call(
        body,
        out_shape=jax.ShapeDtypeStruct((N_TOK, D_FF), jnp.float32),
        in_specs=[
            pl.BlockSpec(memory_space=pltpu.VMEM),
            pl.BlockSpec(memory_space=pltpu.VMEM),
            pl.BlockSpec(memory_space=pltpu.VMEM),
        ],
        out_specs=pl.BlockSpec(memory_space=pltpu.VMEM),
        scratch_shapes=[
            pltpu.VMEM((N_SLOTS, D_FF), jnp.bfloat16),
            pltpu.VMEM((N_DEV - 1, N_SLOTS, D_FF), jnp.bfloat16),
            pltpu.SemaphoreType.DMA((N_DEV - 1,)),
            pltpu.SemaphoreType.DMA((N_DEV - 1,)),
        ],
        compiler_params=pltpu.CompilerParams(collective_id=0),
    )(x, route_idx, expert_W)


# device time: 16159 ns/iter; 5.2151x vs baseline; 1.0580x over previous
import jax
import jax.numpy as jnp
from jax import lax
from jax.experimental import pallas as pl
from jax.experimental.pallas import tpu as pltpu

N_DEV = 4
N_TOK = 1024
D_MODEL = 256
D_FF = 512
EXPERTS_PER_DEV = 4
CAPACITY = 51
SLOT = 52
N_SLOTS = EXPERTS_PER_DEV * SLOT
HALF = D_FF // 2


def kernel(x, router_W, route_idx, expert_W):
    del router_W

    def body(x_ref, idx_ref, w_ref, out_ref,
             y_ref, buf_in, send_sems, recv_sems):
        my_pos = lax.axis_index("i")
        left = lax.rem(my_pos + N_DEV - 1, N_DEV)
        right = lax.rem(my_pos + 1, N_DEV)
        diag = lax.rem(my_pos + 2, N_DEV)

        barrier_sem = pltpu.get_barrier_semaphore()
        for nbr in [left, right, diag]:
            pl.semaphore_signal(
                barrier_sem, inc=1,
                device_id=(nbr,), device_id_type=pl.DeviceIdType.MESH,
            )

        r = idx_ref[:, :]
        e16 = lax.broadcasted_iota(jnp.int32, (N_TOK, 16), 1)
        onehot = (r == e16).astype(jnp.bfloat16)
        NB, BLK = 8, N_TOK // 8
        row = lax.broadcasted_iota(jnp.int32, (NB, BLK, BLK), 1)
        col = lax.broadcasted_iota(jnp.int32, (NB, BLK, BLK), 2)
        tri_b = (col <= row).astype(jnp.bfloat16)
        oh_b = onehot.reshape(NB, BLK, 16)
        cnt_b = lax.dot_general(
            tri_b, oh_b,
            dimension_numbers=(((2,), (1,)), ((0,), (0,))),
            preferred_element_type=jnp.float32,
        )
        totals = cnt_b[:, BLK - 1, :]
        brow = lax.broadcasted_iota(jnp.int32, (NB, NB), 0)
        bcol = lax.broadcasted_iota(jnp.int32, (NB, NB), 1)
        tri_x = (bcol < brow).astype(jnp.bfloat16)
        offs = jnp.dot(
            tri_x, totals.astype(jnp.bfloat16),
            preferred_element_type=jnp.float32,
        )
        cnt16 = (cnt_b + offs[:, None, :]).reshape(N_TOK, 16)
        rk = jnp.sum(
            onehot.astype(jnp.float32) * cnt16, axis=1, keepdims=True
        )

        slot_j = lax.broadcasted_iota(jnp.int32, (N_TOK, N_SLOTS), 1) // SLOT
        slot_r = (
            lax.broadcasted_iota(jnp.int32, (N_TOK, N_SLOTS), 1) % SLOT + 1
        )
        slot_ok = (slot_r <= CAPACITY)

        def gather_t(d):
            slot_e = d * EXPERTS_PER_DEV + slot_j
            return (
                (r == slot_e) & (rk == slot_r.astype(jnp.float32)) & slot_ok
            ).astype(jnp.bfloat16)

        g_me = gather_t(my_pos)
        x_sel = lax.dot_general(
            g_me, x_ref[:, :].astype(jnp.bfloat16),
            dimension_numbers=(((0,), (0,)), ((), ())),
            preferred_element_type=jnp.float32,
        )
        y_ref[:, :] = jnp.concatenate(
            [
                jnp.dot(
                    x_sel[j * SLOT : (j + 1) * SLOT, :].astype(jnp.bfloat16),
                    w_ref[j, :, :].astype(jnp.bfloat16),
                    preferred_element_type=jnp.float32,
                ).astype(jnp.bfloat16)
                for j in range(EXPERTS_PER_DEV)
            ],
            axis=0,
        )

        def push(target, slot, cols=None):
            cd = pl.ds(0, D_FF) if cols is None else cols
            sem = slot if cols is None else slot + 1
            rdma = pltpu.make_async_remote_copy(
                src_ref=y_ref.at[:, cd],
                dst_ref=buf_in.at[slot, :, cd],
                send_sem=send_sems.at[sem],
                recv_sem=recv_sems.at[sem],
                device_id=(target,),
                device_id_type=pl.DeviceIdType.MESH,
            )
            rdma.start()
            return rdma

        cols_a = pl.ds(0, HALF)
        cols_b = pl.ds(HALF, HALF)
        pl.semaphore_wait(barrier_sem, 3)
        sends = [
            push(right, 0),
            push(left, 1),
            push(diag, 2, cols_a),
            push(diag, 2, cols_b),
        ]

        out_ref[:, :] = jnp.dot(
            g_me, y_ref[:, :], preferred_element_type=jnp.float32
        )
        g_peers = [gather_t(left), gather_t(right), gather_t(diag)]

        for slot in range(2):
            sends[slot].wait_recv()
            out_ref[:, :] += jnp.dot(
                g_peers[slot], buf_in[slot, :, :],
                preferred_element_type=jnp.float32,
            )
        for half, cd in [(2, cols_a), (3, cols_b)]:
            sends[half].wait_recv()
            out_ref[:, cd] += jnp.dot(
                g_peers[2], buf_in[2, :, cd],
                preferred_element_type=jnp.float32,
            )
        for s in sends:
            s.wait_send()

    return pl.pallas_call(
        body,
        out_shape=jax.ShapeDtypeStruct((N_TOK, D_FF), jnp.float32),
        in_specs=[
            pl.BlockSpec(memory_space=pltpu.VMEM),
            pl.BlockSpec(memory_space=pltpu.VMEM),
            pl.BlockSpec(memory_space=pltpu.VMEM),
        ],
        out_specs=pl.BlockSpec(memory_space=pltpu.VMEM),
        scratch_shapes=[
            pltpu.VMEM((N_SLOTS, D_FF), jnp.bfloat16),
            pltpu.VMEM((N_DEV - 1, N_SLOTS, D_FF), jnp.bfloat16),
            pltpu.SemaphoreType.DMA((4,)),
            pltpu.SemaphoreType.DMA((4,)),
        ],
        compiler_params=pltpu.CompilerParams(collective_id=0),
    )(x, route_idx, expert_W)
